# Initial kernel scaffold; baseline (speedup 1.0000x reference)
import jax
import jax.numpy as jnp
from jax.experimental import pallas as pl


def kernel(features, adj_info, batch1, batch2, neg_samples, W_self_0, W_neigh_0, W_self_1, W_neigh_1):
    raise NotImplementedError("write your pallas kernel here")


# speedup vs baseline: 1.4253x; 1.4253x over previous
import functools

import jax
import jax.numpy as jnp
from jax import lax
from jax.experimental import pallas as pl
from jax.experimental.pallas import tpu as pltpu
from jax.experimental.pallas import tpu_sc as plsc

D = 128
N_REAL = 1029
NW = 32
IDS_PER_W = 40
N_IDP = NW * IDS_PER_W
S1_PER_W = IDS_PER_W * 10
S1_PAD = 448
N_S1 = N_IDP * 10
FAN1 = 10
FAN2 = 25

C_WIN = 352
C_OUT = 336
C_GROUPS = 68

_sc_mesh = plsc.VectorSubcoreMesh(core_axis_name="c", subcore_axis_name="s")


@functools.partial(
    pl.kernel,
    out_type=[
        jax.ShapeDtypeStruct((N_IDP, D), jnp.float32),
        jax.ShapeDtypeStruct((N_S1, D), jnp.float32),
        jax.ShapeDtypeStruct((N_S1,), jnp.int32),
    ],
    mesh=_sc_mesh,
    compiler_params=pltpu.CompilerParams(needs_layout_passes=False),
    scratch_types=[
        pltpu.VMEM((48,), jnp.int32),
        pltpu.VMEM((48,), jnp.int32),
        pltpu.VMEM((48, 128), jnp.int32),
        pltpu.VMEM((S1_PAD,), jnp.int32),
        pltpu.VMEM((S1_PAD, D), jnp.float32),
        pltpu.VMEM((48, D), jnp.float32),
        pltpu.SemaphoreType.DMA,
    ],
)
def _sc_sample(feat, adjp, ids, h0_out, h1_out, s1_out,
               my_ids, pid, adj0, s1i, h1b, h0b, sem):
    w = lax.axis_index("s") * 2 + lax.axis_index("c")
    iota = lax.iota(jnp.int32, 16)

    pltpu.sync_copy(ids.at[pl.ds(w * IDS_PER_W, IDS_PER_W)],
                    my_ids.at[pl.ds(0, IDS_PER_W)])
    for t in range(3):
        v = plsc.load_gather(my_ids, [jnp.minimum(t * 16 + iota,
                                                  IDS_PER_W - 1)])
        pid[pl.ds(t * 16, 16)] = v >> 2
    pltpu.async_copy(adjp.at[pid], adj0, sem).wait()

    for k in range(S1_PAD // 16):
        j = jnp.minimum(k * 16 + iota, S1_PER_W - 1)
        l = j // FAN1
        mi = plsc.load_gather(my_ids, [l])
        s1i[pl.ds(k * 16, 16)] = plsc.load_gather(
            adj0, [l, (mi & 3) * 32 + j % FAN1])

    for cc in range(4):
        pltpu.async_copy(feat.at[s1i.at[pl.ds(cc * 112, 112)]],
                         h1b.at[pl.ds(cc * 112, 112)], sem).wait()
    pltpu.async_copy(feat.at[my_ids], h0b, sem).wait()

    pltpu.sync_copy(h0b.at[pl.ds(0, IDS_PER_W)],
                    h0_out.at[pl.ds(w * IDS_PER_W, IDS_PER_W)])
    pltpu.sync_copy(h1b.at[pl.ds(0, S1_PER_W)],
                    h1_out.at[pl.ds(w * S1_PER_W, S1_PER_W)])
    pltpu.sync_copy(s1i.at[pl.ds(0, S1_PER_W)],
                    s1_out.at[pl.ds(w * S1_PER_W, S1_PER_W)])


@functools.partial(
    pl.kernel,
    out_type=jax.ShapeDtypeStruct((N_S1, D), jnp.float32),
    mesh=_sc_mesh,
    compiler_params=pltpu.CompilerParams(needs_layout_passes=False),
    scratch_types=[
        pltpu.VMEM((C_WIN,), jnp.int32),
        pltpu.VMEM((C_WIN,), jnp.int32),
        pltpu.VMEM((C_WIN, 128), jnp.int32),
        pltpu.VMEM((128,), jnp.int32),
        pltpu.VMEM((128, D), jnp.float32),
        pltpu.VMEM((C_WIN, D), jnp.float32),
        pltpu.SemaphoreType.DMA,
    ],
)
def _sc_aggregate(feat, adjp, s1l, n1_out, s1c, pid1, adj1, s2i, gbuf,
                  n1b, sem):
    w = lax.axis_index("s") * 2 + lax.axis_index("c")
    iota = lax.iota(jnp.int32, 16)
    start = ((w * 1278) // 31) * 8

    pltpu.sync_copy(s1l.at[pl.ds(start, C_WIN)], s1c)
    for t in range(C_WIN // 16):
        v = plsc.load_gather(s1c, [t * 16 + iota])
        pid1[pl.ds(t * 16, 16)] = v >> 2
    for cc in range(4):
        pltpu.async_copy(adjp.at[pid1.at[pl.ds(cc * 88, 88)]],
                         adj1.at[pl.ds(cc * 88, 88)], sem).wait()

    def group_body(g, carry):
        for c in range(8):
            jl = c * 16 + iota
            node = g * 5 + jl // FAN2
            sv = plsc.load_gather(s1c, [node])
            s2i[pl.ds(c * 16, 16)] = plsc.load_gather(
                adj1, [node, (sv & 3) * 32 + jl % FAN2])
        pltpu.async_copy(feat.at[s2i], gbuf, sem).wait()
        for n in range(5):
            for ch in range(8):
                sl = pl.ds(ch * 16, 16)
                acc = gbuf[n * FAN2, sl]
                for r in range(1, FAN2):
                    acc = acc + gbuf[n * FAN2 + r, sl]
                n1b[g * 5 + n, sl] = acc
        return carry

    lax.fori_loop(0, C_GROUPS, group_body, 0)
    pltpu.sync_copy(n1b.at[pl.ds(0, C_OUT)],
                    n1_out.at[pl.ds(start, C_OUT)])


P = 176
NB = 6


def _tc_body(h0_ref, h1_ref, n1_ref, ws0_ref, wn0_ref, ws1_ref, wn1_ref,
             out_ref):
    f32 = jnp.float32
    h1b = h1_ref[...]
    n1b = n1_ref[...] * f32(1.0 / FAN2)
    ws0 = ws0_ref[...]
    wn0 = wn0_ref[...]
    t1 = jnp.maximum(
        jnp.dot(h1b, ws0, preferred_element_type=f32)
        + jnp.dot(n1b, wn0, preferred_element_type=f32), 0.0)
    rowi = lax.broadcasted_iota(jnp.int32, (P, P * FAN1), 0)
    colg = lax.broadcasted_iota(jnp.int32, (P, P * FAN1), 1) // FAN1
    sel = jnp.where(rowi == colg, f32(1.0 / FAN1), f32(0.0))
    n0 = jnp.dot(sel, h1b, preferred_element_type=f32)
    ntop = jnp.dot(sel, t1, preferred_element_type=f32)
    h0b = h0_ref[...]
    l0 = jnp.maximum(
        jnp.dot(h0b, ws0, preferred_element_type=f32)
        + jnp.dot(n0, wn0, preferred_element_type=f32), 0.0)
    o = (jnp.dot(l0, ws1_ref[...], preferred_element_type=f32)
         + jnp.dot(ntop, wn1_ref[...], preferred_element_type=f32))
    o = o / jnp.sqrt(jnp.maximum(jnp.sum(o * o, axis=1, keepdims=True),
                                 1e-12))
    out_ref[...] = o


def _tc_agg(h0, h1, n1s, ws0, wn0, ws1, wn1):
    wspec = pl.BlockSpec((D, D), lambda i: (0, 0))
    return pl.pallas_call(
        _tc_body,
        grid=(NB,),
        in_specs=[
            pl.BlockSpec((P, D), lambda i: (i, 0)),
            pl.BlockSpec((P * FAN1, D), lambda i: (i, 0)),
            pl.BlockSpec((P * FAN1, D), lambda i: (i, 0)),
            wspec, wspec, wspec, wspec,
        ],
        out_specs=pl.BlockSpec((P, D), lambda i: (i, 0)),
        out_shape=jax.ShapeDtypeStruct((P * NB, D), jnp.float32),
    )(h0, h1, n1s, ws0, wn0, ws1, wn1)


def kernel(features, adj_info, batch1, batch2, neg_samples,
           W_self_0, W_neigh_0, W_self_1, W_neigh_1):
    ids = jnp.concatenate([
        batch1.astype(jnp.int32), batch2.astype(jnp.int32),
        neg_samples.astype(jnp.int32),
        jnp.zeros((N_IDP - N_REAL,), jnp.int32),
    ])
    adjp = adj_info.reshape(adj_info.shape[0] // 4, 128)
    h0, h1, s1 = _sc_sample(features, adjp, ids)
    n1s = _sc_aggregate(features, adjp, s1)
    out = _tc_agg(h0, h1, n1s, W_self_0, W_neigh_0, W_self_1, W_neigh_1)
    return out[:N_REAL]


# speedup vs baseline: 1.6930x; 1.1878x over previous
import functools

import jax
import jax.numpy as jnp
from jax import lax
from jax.experimental import pallas as pl
from jax.experimental.pallas import tpu as pltpu
from jax.experimental.pallas import tpu_sc as plsc

D = 128
N_REAL = 1029
NW = 32
IDS_PER_W = 40
N_IDP = NW * IDS_PER_W
S1_PER_W = IDS_PER_W * 10
S1_PAD = 448
N_S1 = N_IDP * 10
FAN1 = 10
FAN2 = 25

C_WIN = 352
C_OUT = 336
C_GROUPS = 68

_sc_mesh = plsc.VectorSubcoreMesh(core_axis_name="c", subcore_axis_name="s")


@functools.partial(
    pl.kernel,
    out_type=[
        jax.ShapeDtypeStruct((N_IDP, D), jnp.float32),
        jax.ShapeDtypeStruct((N_S1, D), jnp.float32),
        jax.ShapeDtypeStruct((N_S1,), jnp.int32),
    ],
    mesh=_sc_mesh,
    compiler_params=pltpu.CompilerParams(needs_layout_passes=False),
    scratch_types=[
        pltpu.VMEM((48,), jnp.int32),
        pltpu.VMEM((48,), jnp.int32),
        pltpu.VMEM((48, 128), jnp.int32),
        pltpu.VMEM((S1_PAD,), jnp.int32),
        pltpu.VMEM((S1_PAD, D), jnp.float32),
        pltpu.VMEM((48, D), jnp.float32),
        pltpu.SemaphoreType.DMA,
        pltpu.SemaphoreType.DMA,
    ],
)
def _sc_sample(feat, adjp, ids, h0_out, h1_out, s1_out,
               my_ids, pid, adj0, s1i, h1b, h0b, sem, sem2):
    w = lax.axis_index("s") * 2 + lax.axis_index("c")
    iota = lax.iota(jnp.int32, 16)

    pltpu.sync_copy(ids.at[pl.ds(w * IDS_PER_W, IDS_PER_W)],
                    my_ids.at[pl.ds(0, IDS_PER_W)])
    for t in range(3):
        v = plsc.load_gather(my_ids, [jnp.minimum(t * 16 + iota,
                                                  IDS_PER_W - 1)])
        pid[pl.ds(t * 16, 16)] = v >> 2
    pltpu.async_copy(adjp.at[pid], adj0, sem).wait()

    for k in range(S1_PAD // 16):
        j = jnp.minimum(k * 16 + iota, S1_PER_W - 1)
        l = j // FAN1
        mi = plsc.load_gather(my_ids, [l])
        s1i[pl.ds(k * 16, 16)] = plsc.load_gather(
            adj0, [l, (mi & 3) * 32 + j % FAN1])

    copies = [
        pltpu.async_copy(feat.at[s1i.at[pl.ds(cc * 112, 112)]],
                         h1b.at[pl.ds(cc * 112, 112)], sem)
        for cc in range(4)
    ]
    copies.append(pltpu.async_copy(feat.at[my_ids], h0b, sem2))
    for cp in copies:
        cp.wait()

    pltpu.sync_copy(h0b.at[pl.ds(0, IDS_PER_W)],
                    h0_out.at[pl.ds(w * IDS_PER_W, IDS_PER_W)])
    pltpu.sync_copy(h1b.at[pl.ds(0, S1_PER_W)],
                    h1_out.at[pl.ds(w * S1_PER_W, S1_PER_W)])
    pltpu.sync_copy(s1i.at[pl.ds(0, S1_PER_W)],
                    s1_out.at[pl.ds(w * S1_PER_W, S1_PER_W)])


@functools.partial(
    pl.kernel,
    out_type=jax.ShapeDtypeStruct((N_S1, D), jnp.float32),
    mesh=_sc_mesh,
    compiler_params=pltpu.CompilerParams(needs_layout_passes=False),
    scratch_types=[
        pltpu.VMEM((C_WIN,), jnp.int32),
        pltpu.VMEM((C_WIN,), jnp.int32),
        pltpu.VMEM((C_WIN, 128), jnp.int32),
        pltpu.VMEM((128,), jnp.int32),
        pltpu.VMEM((128,), jnp.int32),
        pltpu.VMEM((128, D), jnp.float32),
        pltpu.VMEM((128, D), jnp.float32),
        pltpu.VMEM((C_WIN, D), jnp.float32),
        pltpu.SemaphoreType.DMA,
        pltpu.SemaphoreType.DMA,
    ],
)
def _sc_aggregate(feat, adjp, s1l, n1_out, s1c, pid1, adj1, s2ia, s2ib,
                  gbufa, gbufb, n1b, sema, semb):
    w = lax.axis_index("s") * 2 + lax.axis_index("c")
    iota = lax.iota(jnp.int32, 16)
    start = ((w * 1278) // 31) * 8

    pltpu.sync_copy(s1l.at[pl.ds(start, C_WIN)], s1c)
    for t in range(C_WIN // 16):
        v = plsc.load_gather(s1c, [t * 16 + iota])
        pid1[pl.ds(t * 16, 16)] = v >> 2
    for cc in range(4):
        pltpu.async_copy(adjp.at[pid1.at[pl.ds(cc * 88, 88)]],
                         adj1.at[pl.ds(cc * 88, 88)], sema).wait()

    def build_idx(g, s2i):
        for c in range(8):
            jl = c * 16 + iota
            node = jnp.minimum(g * 5 + jl // FAN2, C_WIN - 1)
            sv = plsc.load_gather(s1c, [node])
            s2i[pl.ds(c * 16, 16)] = plsc.load_gather(
                adj1, [node, (sv & 3) * 32 + jl % FAN2])

    def reduce_group(g, gbuf):
        for n in range(5):
            for ch in range(8):
                sl = pl.ds(ch * 16, 16)
                acc = gbuf[n * FAN2, sl]
                for r in range(1, FAN2):
                    acc = acc + gbuf[n * FAN2 + r, sl]
                n1b[g * 5 + n, sl] = acc

    build_idx(0, s2ia)
    pltpu.async_copy(feat.at[s2ia], gbufa, sema)

    def pair_body(i, carry):
        ga = 2 * i
        build_idx(ga + 1, s2ib)
        pltpu.async_copy(feat.at[s2ib], gbufb, semb)
        pltpu.make_async_copy(feat.at[s2ia], gbufa, sema).wait()
        reduce_group(ga, gbufa)
        build_idx(ga + 2, s2ia)
        pltpu.async_copy(feat.at[s2ia], gbufa, sema)
        pltpu.make_async_copy(feat.at[s2ib], gbufb, semb).wait()
        reduce_group(ga + 1, gbufb)
        return carry

    lax.fori_loop(0, C_GROUPS // 2, pair_body, 0)
    pltpu.make_async_copy(feat.at[s2ia], gbufa, sema).wait()
    pltpu.sync_copy(n1b.at[pl.ds(0, C_OUT)],
                    n1_out.at[pl.ds(start, C_OUT)])


P = 176
NB = 6


def _tc_body(h0_ref, h1_ref, n1_ref, ws0_ref, wn0_ref, ws1_ref, wn1_ref,
             out_ref):
    f32 = jnp.float32
    h1b = h1_ref[...]
    n1b = n1_ref[...] * f32(1.0 / FAN2)
    ws0 = ws0_ref[...]
    wn0 = wn0_ref[...]
    t1 = jnp.maximum(
        jnp.dot(h1b, ws0, preferred_element_type=f32)
        + jnp.dot(n1b, wn0, preferred_element_type=f32), 0.0)
    rowi = lax.broadcasted_iota(jnp.int32, (P, P * FAN1), 0)
    colg = lax.broadcasted_iota(jnp.int32, (P, P * FAN1), 1) // FAN1
    sel = jnp.where(rowi == colg, f32(1.0 / FAN1), f32(0.0))
    n0 = jnp.dot(sel, h1b, preferred_element_type=f32)
    ntop = jnp.dot(sel, t1, preferred_element_type=f32)
    h0b = h0_ref[...]
    l0 = jnp.maximum(
        jnp.dot(h0b, ws0, preferred_element_type=f32)
        + jnp.dot(n0, wn0, preferred_element_type=f32), 0.0)
    o = (jnp.dot(l0, ws1_ref[...], preferred_element_type=f32)
         + jnp.dot(ntop, wn1_ref[...], preferred_element_type=f32))
    o = o / jnp.sqrt(jnp.maximum(jnp.sum(o * o, axis=1, keepdims=True),
                                 1e-12))
    out_ref[...] = o


def _tc_agg(h0, h1, n1s, ws0, wn0, ws1, wn1):
    wspec = pl.BlockSpec((D, D), lambda i: (0, 0))
    return pl.pallas_call(
        _tc_body,
        grid=(NB,),
        in_specs=[
            pl.BlockSpec((P, D), lambda i: (i, 0)),
            pl.BlockSpec((P * FAN1, D), lambda i: (i, 0)),
            pl.BlockSpec((P * FAN1, D), lambda i: (i, 0)),
            wspec, wspec, wspec, wspec,
        ],
        out_specs=pl.BlockSpec((P, D), lambda i: (i, 0)),
        out_shape=jax.ShapeDtypeStruct((P * NB, D), jnp.float32),
    )(h0, h1, n1s, ws0, wn0, ws1, wn1)


def kernel(features, adj_info, batch1, batch2, neg_samples,
           W_self_0, W_neigh_0, W_self_1, W_neigh_1):
    ids = jnp.concatenate([
        batch1.astype(jnp.int32), batch2.astype(jnp.int32),
        neg_samples.astype(jnp.int32),
        jnp.zeros((N_IDP - N_REAL,), jnp.int32),
    ])
    adjp = adj_info.reshape(adj_info.shape[0] // 4, 128)
    h0, h1, s1 = _sc_sample(features, adjp, ids)
    n1s = _sc_aggregate(features, adjp, s1)
    out = _tc_agg(h0, h1, n1s, W_self_0, W_neigh_0, W_self_1, W_neigh_1)
    return out[:N_REAL]


# speedup vs baseline: 2.4121x; 1.4247x over previous
import functools

import jax
import jax.numpy as jnp
from jax import lax
from jax.experimental import pallas as pl
from jax.experimental.pallas import tpu as pltpu
from jax.experimental.pallas import tpu_sc as plsc

D = 128
N_REAL = 1029
NW = 32
IDS_PER_W = 40
N_IDP = NW * IDS_PER_W
S1_PER_W = IDS_PER_W * 10
S1_PAD = 448
N_S1 = N_IDP * 10
FAN1 = 10
FAN2 = 25

C_WIN = 352
C_OUT = 336
C_GROUPS = 68

_sc_mesh = plsc.VectorSubcoreMesh(core_axis_name="c", subcore_axis_name="s")


@functools.partial(
    pl.kernel,
    out_type=[
        jax.ShapeDtypeStruct((N_IDP, D), jnp.float32),
        jax.ShapeDtypeStruct((N_S1, D), jnp.float32),
        jax.ShapeDtypeStruct((N_S1,), jnp.int32),
    ],
    mesh=_sc_mesh,
    compiler_params=pltpu.CompilerParams(needs_layout_passes=False),
    scratch_types=[
        pltpu.VMEM((48,), jnp.int32),
        pltpu.VMEM((48,), jnp.int32),
        pltpu.VMEM((48, 128), jnp.int32),
        pltpu.VMEM((S1_PAD,), jnp.int32),
        pltpu.VMEM((S1_PAD, D), jnp.float32),
        pltpu.VMEM((48, D), jnp.float32),
        pltpu.SemaphoreType.DMA,
        pltpu.SemaphoreType.DMA,
    ],
)
def _sc_sample(feat, adjp, ids, h0_out, h1_out, s1_out,
               my_ids, pid, adj0, s1i, h1b, h0b, sem, sem2):
    w = lax.axis_index("s") * 2 + lax.axis_index("c")
    iota = lax.iota(jnp.int32, 16)

    pltpu.sync_copy(ids.at[pl.ds(w * IDS_PER_W, IDS_PER_W)],
                    my_ids.at[pl.ds(0, IDS_PER_W)])
    for t in range(3):
        v = plsc.load_gather(my_ids, [jnp.minimum(t * 16 + iota,
                                                  IDS_PER_W - 1)])
        pid[pl.ds(t * 16, 16)] = v >> 2
    pltpu.async_copy(adjp.at[pid], adj0, sem).wait()

    for k in range(S1_PAD // 16):
        j = jnp.minimum(k * 16 + iota, S1_PER_W - 1)
        l = j // FAN1
        mi = plsc.load_gather(my_ids, [l])
        s1i[pl.ds(k * 16, 16)] = plsc.load_gather(
            adj0, [l, (mi & 3) * 32 + j % FAN1])

    copies = [
        pltpu.async_copy(feat.at[s1i.at[pl.ds(cc * 112, 112)]],
                         h1b.at[pl.ds(cc * 112, 112)], sem)
        for cc in range(4)
    ]
    copies.append(pltpu.async_copy(feat.at[my_ids], h0b, sem2))
    for cp in copies:
        cp.wait()

    pltpu.sync_copy(h0b.at[pl.ds(0, IDS_PER_W)],
                    h0_out.at[pl.ds(w * IDS_PER_W, IDS_PER_W)])
    pltpu.sync_copy(h1b.at[pl.ds(0, S1_PER_W)],
                    h1_out.at[pl.ds(w * S1_PER_W, S1_PER_W)])
    pltpu.sync_copy(s1i.at[pl.ds(0, S1_PER_W)],
                    s1_out.at[pl.ds(w * S1_PER_W, S1_PER_W)])


@functools.partial(
    pl.kernel,
    out_type=jax.ShapeDtypeStruct((N_S1, D), jnp.float32),
    mesh=_sc_mesh,
    compiler_params=pltpu.CompilerParams(needs_layout_passes=False),
    scratch_types=[
        pltpu.VMEM((C_WIN,), jnp.int32),
        pltpu.VMEM((C_WIN,), jnp.int32),
        pltpu.VMEM((C_WIN, 128), jnp.int32),
        pltpu.VMEM((128,), jnp.int32),
        pltpu.VMEM((128,), jnp.int32),
        pltpu.VMEM((128, D), jnp.float32),
        pltpu.VMEM((128, D), jnp.float32),
        pltpu.VMEM((C_WIN, D), jnp.float32),
        pltpu.SemaphoreType.DMA,
        pltpu.SemaphoreType.DMA,
    ],
)
def _sc_aggregate(feat, adjp, s1l, n1_out, s1c, pid1, adj1, s2ia, s2ib,
                  gbufa, gbufb, n1b, sema, semb):
    w = lax.axis_index("s") * 2 + lax.axis_index("c")
    iota = lax.iota(jnp.int32, 16)
    start = ((w * 1278) // 31) * 8

    pltpu.sync_copy(s1l.at[pl.ds(start, C_WIN)], s1c)
    for t in range(C_WIN // 16):
        v = plsc.load_gather(s1c, [t * 16 + iota])
        pid1[pl.ds(t * 16, 16)] = v >> 2
    for cc in range(4):
        pltpu.async_copy(adjp.at[pid1.at[pl.ds(cc * 88, 88)]],
                         adj1.at[pl.ds(cc * 88, 88)], sema).wait()

    def build_idx(g, s2i):
        for c in range(8):
            jl = c * 16 + iota
            node = jnp.minimum(g * 5 + jl // FAN2, C_WIN - 1)
            sv = plsc.load_gather(s1c, [node])
            s2i[pl.ds(c * 16, 16)] = plsc.load_gather(
                adj1, [node, (sv & 3) * 32 + jl % FAN2])

    def reduce_group(g, gbuf):
        for n in range(5):
            for h in range(2):
                chs = range(4 * h, 4 * h + 4)
                acc = [gbuf[n * FAN2, pl.ds(ch * 16, 16)] for ch in chs]
                for r in range(1, FAN2):
                    for x, ch in enumerate(chs):
                        acc[x] = acc[x] + gbuf[n * FAN2 + r,
                                               pl.ds(ch * 16, 16)]
                for x, ch in enumerate(chs):
                    n1b[g * 5 + n, pl.ds(ch * 16, 16)] = acc[x]

    build_idx(0, s2ia)
    pltpu.async_copy(feat.at[s2ia], gbufa, sema)

    def pair_body(i, carry):
        ga = 2 * i
        build_idx(ga + 1, s2ib)
        pltpu.async_copy(feat.at[s2ib], gbufb, semb)
        pltpu.make_async_copy(feat.at[s2ia], gbufa, sema).wait()
        reduce_group(ga, gbufa)
        build_idx(ga + 2, s2ia)
        pltpu.async_copy(feat.at[s2ia], gbufa, sema)
        pltpu.make_async_copy(feat.at[s2ib], gbufb, semb).wait()
        reduce_group(ga + 1, gbufb)
        return carry

    lax.fori_loop(0, C_GROUPS // 2, pair_body, 0)
    pltpu.make_async_copy(feat.at[s2ia], gbufa, sema).wait()
    pltpu.sync_copy(n1b.at[pl.ds(0, C_OUT)],
                    n1_out.at[pl.ds(start, C_OUT)])


P = 176
NB = 6


def _tc_body(h0_ref, h1_ref, n1_ref, ws0_ref, wn0_ref, ws1_ref, wn1_ref,
             out_ref):
    f32 = jnp.float32
    h1b = h1_ref[...]
    n1b = n1_ref[...] * f32(1.0 / FAN2)
    ws0 = ws0_ref[...]
    wn0 = wn0_ref[...]
    t1 = jnp.maximum(
        jnp.dot(h1b, ws0, preferred_element_type=f32)
        + jnp.dot(n1b, wn0, preferred_element_type=f32), 0.0)
    rowi = lax.broadcasted_iota(jnp.int32, (P, P * FAN1), 0)
    colg = lax.broadcasted_iota(jnp.int32, (P, P * FAN1), 1) // FAN1
    sel = jnp.where(rowi == colg, f32(1.0 / FAN1), f32(0.0))
    n0 = jnp.dot(sel, h1b, preferred_element_type=f32)
    ntop = jnp.dot(sel, t1, preferred_element_type=f32)
    h0b = h0_ref[...]
    l0 = jnp.maximum(
        jnp.dot(h0b, ws0, preferred_element_type=f32)
        + jnp.dot(n0, wn0, preferred_element_type=f32), 0.0)
    o = (jnp.dot(l0, ws1_ref[...], preferred_element_type=f32)
         + jnp.dot(ntop, wn1_ref[...], preferred_element_type=f32))
    o = o / jnp.sqrt(jnp.maximum(jnp.sum(o * o, axis=1, keepdims=True),
                                 1e-12))
    out_ref[...] = o


def _tc_agg(h0, h1, n1s, ws0, wn0, ws1, wn1):
    wspec = pl.BlockSpec((D, D), lambda i: (0, 0))
    return pl.pallas_call(
        _tc_body,
        grid=(NB,),
        in_specs=[
            pl.BlockSpec((P, D), lambda i: (i, 0)),
            pl.BlockSpec((P * FAN1, D), lambda i: (i, 0)),
            pl.BlockSpec((P * FAN1, D), lambda i: (i, 0)),
            wspec, wspec, wspec, wspec,
        ],
        out_specs=pl.BlockSpec((P, D), lambda i: (i, 0)),
        out_shape=jax.ShapeDtypeStruct((P * NB, D), jnp.float32),
    )(h0, h1, n1s, ws0, wn0, ws1, wn1)


def kernel(features, adj_info, batch1, batch2, neg_samples,
           W_self_0, W_neigh_0, W_self_1, W_neigh_1):
    ids = jnp.concatenate([
        batch1.astype(jnp.int32), batch2.astype(jnp.int32),
        neg_samples.astype(jnp.int32),
        jnp.zeros((N_IDP - N_REAL,), jnp.int32),
    ])
    adjp = adj_info.reshape(adj_info.shape[0] // 4, 128)
    h0, h1, s1 = _sc_sample(features, adjp, ids)
    n1s = _sc_aggregate(features, adjp, s1)
    out = _tc_agg(h0, h1, n1s, W_self_0, W_neigh_0, W_self_1, W_neigh_1)
    return out[:N_REAL]
